# Initial kernel scaffold; baseline (speedup 1.0000x reference)
import functools

import jax
import jax.numpy as jnp
from jax.experimental import pallas as pl
from jax.experimental.pallas import tpu as pltpu

_B, _T, _C = 2, 2048, 768
_E = 64
_DFF = 768
_N = _B * _T
_K = 128


def _moe_body(sel_smem, x_ref, w1_ref, w2_ref, wt_ref, out_ref, xs_ref, cs_ref):
    e = pl.program_id(0)

    @pl.when(e == 0)
    def _init():
        out_ref[...] = jnp.zeros_like(out_ref)

    def _gather(i, _):
        t = sel_smem[e * _K + i]
        xs_ref[pl.ds(i, 1), :] = x_ref[pl.ds(t, 1), :]
        return 0

    jax.lax.fori_loop(0, _K, _gather, 0, unroll=8)

    h = jnp.dot(xs_ref[...], w1_ref[0], preferred_element_type=jnp.float32)
    h = jax.nn.gelu(h, approximate=False)
    out = jnp.dot(h, w2_ref[0], preferred_element_type=jnp.float32)
    w_col = wt_ref[:, pl.ds(e, 1)]
    cs_ref[...] = out * w_col

    def _scatter(i, _):
        t = sel_smem[e * _K + i]
        out_ref[pl.ds(t, 1), :] = out_ref[pl.ds(t, 1), :] + cs_ref[pl.ds(i, 1), :]
        return 0

    jax.lax.fori_loop(0, _K, _scatter, 0, unroll=8)


@functools.partial(jax.jit, static_argnames=("interpret",))
def _moe_pallas(x2d, w1, w2, weights_t, sel_flat, interpret=False):
    grid_spec = pltpu.PrefetchScalarGridSpec(
        num_scalar_prefetch=1,
        grid=(_E,),
        in_specs=[
            pl.BlockSpec((_N, _C), lambda e, sel: (0, 0)),
            pl.BlockSpec((1, _C, _DFF), lambda e, sel: (e, 0, 0)),
            pl.BlockSpec((1, _DFF, _C), lambda e, sel: (e, 0, 0)),
            pl.BlockSpec((_K, _E), lambda e, sel: (0, 0)),
        ],
        out_specs=pl.BlockSpec((_N, _C), lambda e, sel: (0, 0)),
        scratch_shapes=[
            pltpu.VMEM((_K, _C), jnp.float32),
            pltpu.VMEM((_K, _C), jnp.float32),
        ],
    )
    return pl.pallas_call(
        _moe_body,
        grid_spec=grid_spec,
        out_shape=jax.ShapeDtypeStruct((_N, _C), jnp.float32),
        compiler_params=pltpu.CompilerParams(
            dimension_semantics=("arbitrary",),
        ),
        interpret=interpret,
    )(sel_flat, x2d, w1, w2, weights_t)


def kernel(x, Wr, W1, W2):
    x2d = x.reshape(-1, _C)
    router_logits = x2d @ Wr.T
    probs = jax.nn.softmax(router_logits.astype(jnp.float32), axis=-1)
    weights, sel = jax.lax.top_k(probs.T, _K)
    results = _moe_pallas(
        x2d, W1, W2, weights.T, sel.reshape(-1).astype(jnp.int32)
    )
    return results.reshape(x.shape), router_logits, sel


# speedup vs baseline: 1.6550x; 1.6550x over previous
import functools

import jax
import jax.numpy as jnp
from jax.experimental import pallas as pl
from jax.experimental.pallas import tpu as pltpu

_B, _T, _C = 2, 2048, 768
_E = 64
_DFF = 768
_N = _B * _T
_K = 128


def _moe_body(sel_smem, x_ref, w1_ref, w2_ref, wt_ref, out_ref, xs_ref, cs_ref):
    e = pl.program_id(0)

    @pl.when(e == 0)
    def _init():
        out_ref[...] = jnp.zeros_like(out_ref)

    def _gather(i, _):
        t = sel_smem[e * _K + i]
        xs_ref[pl.ds(i, 1), :] = x_ref[pl.ds(t, 1), :]
        return 0

    jax.lax.fori_loop(0, _K, _gather, 0, unroll=8)

    h = jnp.dot(xs_ref[...], w1_ref[0], preferred_element_type=jnp.float32)
    h = 0.5 * h * (1.0 + jax.lax.erf(h * 0.7071067811865476))
    out = jnp.dot(h, w2_ref[0], preferred_element_type=jnp.float32)
    lane = jax.lax.broadcasted_iota(jnp.int32, (_K, _E), 1)
    w_col = jnp.sum(jnp.where(lane == e, wt_ref[...], 0.0), axis=1, keepdims=True)
    cs_ref[...] = out * w_col

    def _scatter(i, _):
        t = sel_smem[e * _K + i]
        out_ref[pl.ds(t, 1), :] = out_ref[pl.ds(t, 1), :] + cs_ref[pl.ds(i, 1), :]
        return 0

    jax.lax.fori_loop(0, _K, _scatter, 0, unroll=8)


@functools.partial(jax.jit, static_argnames=("interpret",))
def _moe_pallas(x2d, w1, w2, weights_t, sel_flat, interpret=False):
    grid_spec = pltpu.PrefetchScalarGridSpec(
        num_scalar_prefetch=1,
        grid=(_E,),
        in_specs=[
            pl.BlockSpec((_N, _C), lambda e, sel: (0, 0)),
            pl.BlockSpec((1, _C, _DFF), lambda e, sel: (e, 0, 0)),
            pl.BlockSpec((1, _DFF, _C), lambda e, sel: (e, 0, 0)),
            pl.BlockSpec((_K, _E), lambda e, sel: (0, 0)),
        ],
        out_specs=pl.BlockSpec((_N, _C), lambda e, sel: (0, 0)),
        scratch_shapes=[
            pltpu.VMEM((_K, _C), jnp.float32),
            pltpu.VMEM((_K, _C), jnp.float32),
        ],
    )
    return pl.pallas_call(
        _moe_body,
        grid_spec=grid_spec,
        out_shape=jax.ShapeDtypeStruct((_N, _C), jnp.float32),
        compiler_params=pltpu.CompilerParams(
            dimension_semantics=("arbitrary",),
        ),
        interpret=interpret,
    )(sel_flat, x2d, w1, w2, weights_t)


def kernel(x, Wr, W1, W2):
    x2d = x.reshape(-1, _C)
    router_logits = x2d @ Wr.T
    probs = jax.nn.softmax(router_logits.astype(jnp.float32), axis=-1)
    weights, sel = jax.lax.top_k(probs.T, _K)
    results = _moe_pallas(
        x2d, W1, W2, weights.T, sel.reshape(-1).astype(jnp.int32)
    )
    return results.reshape(x.shape), router_logits, sel


# speedup vs baseline: 1.9073x; 1.1524x over previous
import functools

import jax
import jax.numpy as jnp
from jax.experimental import pallas as pl
from jax.experimental.pallas import tpu as pltpu

_B, _T, _C = 2, 2048, 768
_E = 64
_DFF = 768
_N = _B * _T
_K = 128


def _moe_body(sel_smem, x_ref, w1_ref, w2_ref, wt_ref, out_ref, xs_ref, cs_ref):
    e = pl.program_id(0)

    @pl.when(e == 0)
    def _init():
        out_ref[...] = jnp.zeros_like(out_ref)

    def _gather(i, _):
        t = sel_smem[e * _K + i]
        xs_ref[pl.ds(i, 1), :] = x_ref[pl.ds(t, 1), :]
        return 0

    jax.lax.fori_loop(0, _K, _gather, 0, unroll=8)

    h = jnp.dot(xs_ref[...], w1_ref[0], preferred_element_type=jnp.float32)
    h = 0.5 * h * (1.0 + jax.lax.erf(h * 0.7071067811865476))
    out = jnp.dot(h, w2_ref[0], preferred_element_type=jnp.float32)
    lane = jax.lax.broadcasted_iota(jnp.int32, (_K, _E), 1)
    w_col = jnp.sum(jnp.where(lane == e, wt_ref[...], 0.0), axis=1, keepdims=True)
    cs_ref[...] = out * w_col

    def _scatter(i, _):
        t = sel_smem[e * _K + i]
        out_ref[pl.ds(t, 1), :] = out_ref[pl.ds(t, 1), :] + cs_ref[pl.ds(i, 1), :]
        return 0

    jax.lax.fori_loop(0, _K, _scatter, 0, unroll=8)


@functools.partial(jax.jit, static_argnames=("interpret",))
def _moe_pallas(x2d, w1, w2, weights_t, sel_flat, interpret=False):
    grid_spec = pltpu.PrefetchScalarGridSpec(
        num_scalar_prefetch=1,
        grid=(_E,),
        in_specs=[
            pl.BlockSpec((_N, _C), lambda e, sel: (0, 0)),
            pl.BlockSpec((1, _C, _DFF), lambda e, sel: (e, 0, 0)),
            pl.BlockSpec((1, _DFF, _C), lambda e, sel: (e, 0, 0)),
            pl.BlockSpec((_K, _E), lambda e, sel: (0, 0)),
        ],
        out_specs=pl.BlockSpec((_N, _C), lambda e, sel: (0, 0)),
        scratch_shapes=[
            pltpu.VMEM((_K, _C), jnp.float32),
            pltpu.VMEM((_K, _C), jnp.float32),
        ],
    )
    return pl.pallas_call(
        _moe_body,
        grid_spec=grid_spec,
        out_shape=jax.ShapeDtypeStruct((_N, _C), jnp.float32),
        compiler_params=pltpu.CompilerParams(
            dimension_semantics=("arbitrary",),
        ),
        interpret=interpret,
    )(sel_flat, x2d, w1, w2, weights_t)


def kernel(x, Wr, W1, W2):
    x2d = x.reshape(-1, _C)
    router_logits = x2d @ Wr.T
    probs = jax.nn.softmax(router_logits.astype(jnp.float32), axis=-1)
    pt = probs.T
    cv, ci = jax.lax.top_k(pt.reshape(_E * 8, _N // 8), _K)
    cv = cv.reshape(_E, 8 * _K)
    ci = (ci.reshape(_E, 8, _K)
          + (jnp.arange(8, dtype=ci.dtype) * (_N // 8))[None, :, None]
          ).reshape(_E, 8 * _K)
    weights, pos = jax.lax.top_k(cv, _K)
    sel = jnp.take_along_axis(ci, pos, axis=1)
    results = _moe_pallas(
        x2d, W1, W2, weights.T, sel.reshape(-1).astype(jnp.int32)
    )
    return results.reshape(x.shape), router_logits, sel
